# Initial kernel scaffold; baseline (speedup 1.0000x reference)
import jax
import jax.numpy as jnp
from jax.experimental import pallas as pl


def kernel(x_bits, conn_state, hash_coeff, conn_out, ram_state, ram_out):
    raise NotImplementedError("write your pallas kernel here")


# speedup vs baseline: 26.5590x; 26.5590x over previous
import functools

import jax
import jax.numpy as jnp
import numpy as np
from jax import lax
from jax.experimental import pallas as pl
from jax.experimental.pallas import tpu as pltpu
from jax.experimental.pallas import tpu_sc as plsc

H = 16
INPUT_BITS = 24
N_STATE = 16
N_OUT = 16
N_BITS_OUT = 5
HASH = 65536
K_BITS = 4
T = 8192
C = INPUT_BITS + N_STATE

TB = 512
CH = 512
TC_PER_W = T // 32


def _phase_a_body(x_ref, w_ref, ain_ref, key_ref):
    x = x_ref[...]
    w = w_ref[0]
    acc = jnp.zeros((TB, N_STATE), jnp.int32)
    for b in range(INPUT_BITS):
        acc = acc + x[:, b:b + 1] * w[b:b + 1, :]
    ain_ref[0] = acc & (HASH - 1)
    k = x[:, 0:1] * 8 + x[:, 1:2] * 4 + x[:, 2:3] * 2 + x[:, 3:4]
    key_ref[...] = jnp.broadcast_to(k, (TB, N_STATE))


def _phase_a(x_bits, win_t):
    return pl.pallas_call(
        _phase_a_body,
        grid=(T // TB, H),
        in_specs=[
            pl.BlockSpec((TB, INPUT_BITS), lambda t, h: (t, 0)),
            pl.BlockSpec((1, INPUT_BITS, N_STATE), lambda t, h: (h, 0, 0)),
        ],
        out_specs=[
            pl.BlockSpec((1, TB, N_STATE), lambda t, h: (h, t, 0)),
            pl.BlockSpec((TB, N_STATE), lambda t, h: (t, 0)),
        ],
        out_shape=[
            jax.ShapeDtypeStruct((H, T, N_STATE), jnp.int32),
            jax.ShapeDtypeStruct((T, N_STATE), jnp.int32),
        ],
    )(x_bits, win_t)


def _iota16():
    return lax.iota(jnp.int32, 16)


def _bcast_lane(vec, lane):
    idx = jnp.full((16,), lane, jnp.int32)
    return vec.at[idx].get(mode="promise_in_bounds")


def _phase_b_body(ain_hbm, sp_hbm, ws_hbm, addr_hbm, sp_v, ws_v, ain_v, addr_v):
    c = lax.axis_index("c")
    s = lax.axis_index("s")
    h = c * 8 + s

    @pl.when(s < 8)
    def _():
        pltpu.sync_copy(sp_hbm.at[h], sp_v)
        pltpu.sync_copy(ws_hbm.at[h], ws_v)
        nbase = _iota16() * 2048

        def step(i, bits):
            acc = jnp.zeros((16,), jnp.int32)
            for si in range(N_STATE):
                acc = acc + ws_v[pl.ds(si * 16, 16)] * _bcast_lane(bits, si)
            a = (ain_v[pl.ds(i * 16, 16)] + acc) & (HASH - 1)
            w = plsc.load_gather(sp_v, [(a & 2047) + nbase])
            bit = (w >> (a >> 11)) & 1
            addr_v[pl.ds(i * 16, 16)] = a | (bit << 16)
            return bit

        def chunk(ci, bits):
            pltpu.sync_copy(ain_hbm.at[h, pl.ds(ci * CH * 16, CH * 16)], ain_v)
            bits = lax.fori_loop(0, CH, step, bits, unroll=2)
            pltpu.sync_copy(addr_v, addr_hbm.at[h, pl.ds(ci * CH * 16, CH * 16)])
            return bits

        lax.fori_loop(0, T // CH, chunk, jnp.zeros((16,), jnp.int32))


def _phase_b(ain_flat, sp_flat, ws_flat):
    kern = pl.kernel(
        _phase_b_body,
        out_type=jax.ShapeDtypeStruct((H, T * 16), jnp.int32),
        mesh=plsc.VectorSubcoreMesh(core_axis_name="c", subcore_axis_name="s"),
        compiler_params=pltpu.CompilerParams(needs_layout_passes=False),
        scratch_types=[
            pltpu.VMEM((N_STATE * 2048,), jnp.int32),
            pltpu.VMEM((N_STATE * 16,), jnp.int32),
            pltpu.VMEM((CH * 16,), jnp.int32),
            pltpu.VMEM((CH * 16,), jnp.int32),
        ],
    )
    return kern(ain_flat, sp_flat, ws_flat)


def _phase_c_body(addr_hbm, key_hbm, ram_hbm, wout_hbm, wcnt_hbm,
                  rout_hbm, out_hbm, key_v, ridx_v, addr_v, gidx_v, raw_v,
                  wout_v, wcnt_v, rout_v, out_v, sem):
    c = lax.axis_index("c")
    s = lax.axis_index("s")
    wid = s * 2 + c
    t0 = wid * TC_PER_W
    iota = _iota16()

    pltpu.sync_copy(wout_hbm, wout_v)
    pltpu.sync_copy(wcnt_hbm, wcnt_v)
    pltpu.sync_copy(rout_hbm, rout_v)
    pltpu.sync_copy(key_hbm.at[pl.ds(t0 * 16, TC_PER_W * 16)], key_v)

    def mk_ridx(g, _):
        kvec = plsc.load_gather(key_v, [iota * 16 + g * 256])
        ridx_v[pl.ds(g * 16, 16)] = kvec * T + (t0 + g * 16 + iota)
        return 0

    lax.fori_loop(0, TC_PER_W // 16, mk_ridx, 0)

    def fetch_addr(g, _):
        pltpu.async_copy(
            addr_hbm.at[ridx_v.at[pl.ds(g * 128, 128)]],
            addr_v.at[pl.ds(g * 128, 128)], sem).wait()
        return 0

    lax.fori_loop(0, TC_PER_W // 128, fetch_addr, 0)

    def mk_gidx(i, _):
        k_s = key_v[pl.ds(i * 16, 16)]
        a = plsc.load_gather(addr_v, [jnp.full((16,), i, jnp.int32), iota])
        gidx_v[pl.ds(i * 16, 16)] = (k_s * 16 + iota) * HASH + (a & (HASH - 1))
        return 0

    lax.fori_loop(0, TC_PER_W, mk_gidx, 0)

    n_g = TC_PER_W * 16 // 128

    def fetch_raw(j, _):
        pltpu.async_copy(ram_hbm.at[gidx_v.at[pl.ds(j * 128, 128)]],
                         raw_v.at[pl.ds(j * 128, 128)], sem).wait()
        return 0

    lax.fori_loop(0, n_g, fetch_raw, 0)

    def emit(i, _):
        k_s = key_v[pl.ds(i * 16, 16)]
        v = plsc.load_gather(addr_v, [jnp.full((16,), i, jnp.int32), iota])
        hard = (v >> 16) & 1
        raw = raw_v[pl.ds(i * 16, 16)]
        sig = 1.0 / (1.0 + jnp.exp(-raw))
        acc_a = jnp.zeros((16,), jnp.int32)
        acc_c = jnp.zeros((16,), jnp.float32)
        base = k_s * 256 + iota
        for si in range(N_STATE):
            wo = plsc.load_gather(wout_v, [base + si * 16])
            wc = plsc.load_gather(wcnt_v, [base + si * 16])
            acc_a = acc_a + wo * _bcast_lane(hard, si)
            acc_c = acc_c + wc * _bcast_lane(sig, si)
        oidx = k_s * 512 + iota * 32 + acc_a
        lut = plsc.load_gather(rout_v, [oidx])
        out_v[pl.ds(i * 16, 16)] = lut * (0.5 + acc_c * (1.0 / N_BITS_OUT))
        return 0

    lax.fori_loop(0, TC_PER_W, emit, 0)
    pltpu.sync_copy(out_v, out_hbm.at[pl.ds(t0 * 16, TC_PER_W * 16)])


def _phase_c(addr_rows, key_flat, ram_flat, wout_flat, wcnt_flat, rout_flat):
    kern = pl.kernel(
        _phase_c_body,
        out_type=jax.ShapeDtypeStruct((T * 16,), jnp.float32),
        mesh=plsc.VectorSubcoreMesh(core_axis_name="c", subcore_axis_name="s"),
        compiler_params=pltpu.CompilerParams(needs_layout_passes=False,
                                             use_tc_tiling_on_sc=False),
        scratch_types=[
            pltpu.VMEM((TC_PER_W * 16,), jnp.int32),
            pltpu.VMEM((TC_PER_W,), jnp.int32),
            pltpu.VMEM((TC_PER_W, 16), jnp.int32),
            pltpu.VMEM((TC_PER_W * 16,), jnp.int32),
            pltpu.VMEM((TC_PER_W * 16,), jnp.float32),
            pltpu.VMEM((H * 256,), jnp.int32),
            pltpu.VMEM((H * 256,), jnp.float32),
            pltpu.VMEM((H * N_OUT * 32,), jnp.float32),
            pltpu.VMEM((TC_PER_W * 16,), jnp.float32),
            pltpu.SemaphoreType.DMA,
        ],
    )
    return kern(addr_rows, key_flat, ram_flat, wout_flat, wcnt_flat, rout_flat)


@jax.jit
def kernel(x_bits, conn_state, hash_coeff, conn_out, ram_state, ram_out):
    oh = (conn_state[..., None] == jnp.arange(C)[None, None, None, :])
    w_all = jnp.sum(hash_coeff[..., None] * oh.astype(jnp.int32), axis=2)
    win_t = jnp.transpose(w_all[:, :, :INPUT_BITS] % HASH, (0, 2, 1))
    ws_t = jnp.transpose(w_all[:, :, INPUT_BITS:] % HASH, (0, 2, 1))
    pow2 = jnp.asarray(2 ** np.arange(N_BITS_OUT - 1, -1, -1), dtype=jnp.int32)
    oh_out = (conn_out[..., None] == jnp.arange(N_STATE)[None, None, None, :])
    wout = jnp.sum(pow2[None, None, :, None] * oh_out.astype(jnp.int32), axis=2)
    wout_t = jnp.transpose(wout, (0, 2, 1)).reshape(-1)
    wcnt_t = jnp.transpose(jnp.sum(oh_out, axis=2).astype(jnp.float32),
                           (0, 2, 1)).reshape(-1)

    sgn = (jax.nn.sigmoid(ram_state) > 0.5).astype(jnp.int32)
    sgn = sgn.reshape(H, N_STATE, 32, 2048)
    sp = jnp.sum(sgn << jnp.arange(32, dtype=jnp.int32)[None, None, :, None],
                 axis=2)
    sp_flat = sp.reshape(H, N_STATE * 2048)

    ain, key = _phase_a(x_bits, win_t)

    addr_hist = _phase_b(ain.reshape(H, T * 16), sp_flat,
                         ws_t.reshape(H, N_STATE * 16))

    out = _phase_c(addr_hist.reshape(H * T, 16), key.reshape(T * 16),
                   ram_state.reshape(-1), wout_t, wcnt_t, ram_out.reshape(-1))
    return out.reshape(T, N_OUT)


# speedup vs baseline: 61.0047x; 2.2969x over previous
import jax
import jax.numpy as jnp
import numpy as np
from jax import lax
from jax.experimental import pallas as pl
from jax.experimental.pallas import tpu as pltpu
from jax.experimental.pallas import tpu_sc as plsc

H = 16
INPUT_BITS = 24
N_STATE = 16
N_OUT = 16
N_BITS_OUT = 5
HASH = 65536
K_BITS = 4
T = 8192
C = INPUT_BITS + N_STATE

TB = 512
CH = 512
TC_PER_W = T // 32


def _phase_a_body(x_ref, w_ref, ain_ref, key_ref):
    x = x_ref[...]
    acc = jnp.zeros((TB, H * N_STATE), jnp.int32)
    for b in range(INPUT_BITS):
        acc = acc + x[:, b:b + 1] * w_ref[b:b + 1, :]
    ain_ref[...] = acc & (HASH - 1)
    k = x[:, 0:1] * 8 + x[:, 1:2] * 4 + x[:, 2:3] * 2 + x[:, 3:4]
    key_ref[...] = jnp.broadcast_to(k, (TB, 128))


def _phase_a(x_bits, w2):
    return pl.pallas_call(
        _phase_a_body,
        grid=(T // TB,),
        in_specs=[
            pl.BlockSpec((TB, INPUT_BITS), lambda t: (t, 0)),
            pl.BlockSpec((INPUT_BITS, H * N_STATE), lambda t: (0, 0)),
        ],
        out_specs=[
            pl.BlockSpec((TB, H * N_STATE), lambda t: (t, 0)),
            pl.BlockSpec((TB, 128), lambda t: (t, 0)),
        ],
        out_shape=[
            jax.ShapeDtypeStruct((T, H * N_STATE), jnp.int32),
            jax.ShapeDtypeStruct((T, 128), jnp.int32),
        ],
    )(x_bits, w2)


def _iota16():
    return lax.iota(jnp.int32, 16)


def _bcast_lane(vec, lane):
    idx = jnp.full((16,), lane, jnp.int32)
    return vec.at[idx].get(mode="promise_in_bounds")


def _phase_b_body(ain_hbm, sp_hbm, ws_hbm, addr_hbm, sp_v, ws_v, ain_v, addr_v):
    c = lax.axis_index("c")
    s = lax.axis_index("s")
    h = c * 8 + s

    @pl.when(s < 8)
    def _():
        pltpu.sync_copy(sp_hbm.at[h], sp_v)
        pltpu.sync_copy(ws_hbm.at[h], ws_v)
        iota = _iota16()
        nbase = iota * 2048

        def step(i, bits):
            acc = jnp.zeros((16,), jnp.int32)
            for si in range(N_STATE):
                acc = acc + ws_v[pl.ds(si * 16, 16)] * _bcast_lane(bits, si)
            row = jnp.full((16,), i, jnp.int32)
            a = (plsc.load_gather(ain_v, [row, iota]) + acc) & (HASH - 1)
            w = plsc.load_gather(sp_v, [(a & 2047) + nbase])
            bit = (w >> (a >> 11)) & 1
            plsc.store_scatter(addr_v, [row, iota], a | (bit << 16))
            return bit

        def chunk(ci, bits):
            pltpu.sync_copy(
                ain_hbm.at[pl.ds(ci * CH, CH), pl.ds(h * 16, 16)], ain_v)
            bits = lax.fori_loop(0, CH, step, bits, unroll=2)
            pltpu.sync_copy(addr_v, addr_hbm.at[pl.ds(h * T + ci * CH, CH), :])
            return bits

        lax.fori_loop(0, T // CH, chunk, jnp.zeros((16,), jnp.int32))


def _phase_b(ain2, sp_flat, ws_flat):
    kern = pl.kernel(
        _phase_b_body,
        out_type=jax.ShapeDtypeStruct((H * T, 16), jnp.int32),
        mesh=plsc.VectorSubcoreMesh(core_axis_name="c", subcore_axis_name="s"),
        compiler_params=pltpu.CompilerParams(needs_layout_passes=False,
                                             use_tc_tiling_on_sc=False),
        scratch_types=[
            pltpu.VMEM((N_STATE * 2048,), jnp.int32),
            pltpu.VMEM((N_STATE * 16,), jnp.int32),
            pltpu.VMEM((CH, 16), jnp.int32),
            pltpu.VMEM((CH, 16), jnp.int32),
        ],
    )
    return kern(ain2, sp_flat, ws_flat)


def _phase_c_body(addr_hbm, key_hbm, ram_hbm, wout_hbm, wcnt_hbm,
                  rout_hbm, out_hbm, key_v, ridx_v, addr_v, gidx_v, raw_v,
                  wout_v, wcnt_v, rout_v, out_v, sem):
    c = lax.axis_index("c")
    s = lax.axis_index("s")
    wid = s * 2 + c
    t0 = wid * TC_PER_W
    iota = _iota16()
    zeros = jnp.zeros((16,), jnp.int32)

    pltpu.sync_copy(wout_hbm, wout_v)
    pltpu.sync_copy(wcnt_hbm, wcnt_v)
    pltpu.sync_copy(rout_hbm, rout_v)
    pltpu.sync_copy(key_hbm.at[pl.ds(t0, TC_PER_W), pl.ds(0, 16)], key_v)

    def mk_ridx(g, _):
        kvec = plsc.load_gather(key_v, [g * 16 + iota, zeros])
        ridx_v[pl.ds(g * 16, 16)] = kvec * T + (t0 + g * 16 + iota)
        return 0

    lax.fori_loop(0, TC_PER_W // 16, mk_ridx, 0)

    def fetch_addr(g, _):
        pltpu.async_copy(
            addr_hbm.at[ridx_v.at[pl.ds(g * 128, 128)]],
            addr_v.at[pl.ds(g * 128, 128)], sem).wait()
        return 0

    lax.fori_loop(0, TC_PER_W // 128, fetch_addr, 0)

    def mk_gidx(i, _):
        row = jnp.full((16,), i, jnp.int32)
        k_s = plsc.load_gather(key_v, [row, iota])
        a = plsc.load_gather(addr_v, [row, iota])
        gidx_v[pl.ds(i * 16, 16)] = (k_s * 16 + iota) * HASH + (a & (HASH - 1))
        return 0

    lax.fori_loop(0, TC_PER_W, mk_gidx, 0)

    n_g = TC_PER_W * 16 // 128

    def fetch_raw(j, _):
        pltpu.async_copy(ram_hbm.at[gidx_v.at[pl.ds(j * 128, 128)]],
                         raw_v.at[pl.ds(j * 128, 128)], sem).wait()
        return 0

    lax.fori_loop(0, n_g, fetch_raw, 0)

    def emit(i, _):
        row = jnp.full((16,), i, jnp.int32)
        k_s = plsc.load_gather(key_v, [row, iota])
        v = plsc.load_gather(addr_v, [row, iota])
        hard = (v >> 16) & 1
        raw = raw_v[pl.ds(i * 16, 16)]
        sig = 1.0 / (1.0 + jnp.exp(-raw))
        acc_a = jnp.zeros((16,), jnp.int32)
        acc_c = jnp.zeros((16,), jnp.float32)
        base = k_s * 256 + iota
        for si in range(N_STATE):
            wo = plsc.load_gather(wout_v, [base + si * 16])
            wc = plsc.load_gather(wcnt_v, [base + si * 16])
            acc_a = acc_a + wo * _bcast_lane(hard, si)
            acc_c = acc_c + wc * _bcast_lane(sig, si)
        oidx = k_s * 512 + iota * 32 + acc_a
        lut = plsc.load_gather(rout_v, [oidx])
        plsc.store_scatter(out_v, [row, iota],
                           lut * (0.5 + acc_c * (1.0 / N_BITS_OUT)))
        return 0

    lax.fori_loop(0, TC_PER_W, emit, 0)
    pltpu.sync_copy(out_v, out_hbm.at[pl.ds(t0, TC_PER_W), :])


def _phase_c(addr_rows, key2, ram_flat, wout_flat, wcnt_flat, rout_flat):
    kern = pl.kernel(
        _phase_c_body,
        out_type=jax.ShapeDtypeStruct((T, 16), jnp.float32),
        mesh=plsc.VectorSubcoreMesh(core_axis_name="c", subcore_axis_name="s"),
        compiler_params=pltpu.CompilerParams(needs_layout_passes=False,
                                             use_tc_tiling_on_sc=False),
        scratch_types=[
            pltpu.VMEM((TC_PER_W, 16), jnp.int32),
            pltpu.VMEM((TC_PER_W,), jnp.int32),
            pltpu.VMEM((TC_PER_W, 16), jnp.int32),
            pltpu.VMEM((TC_PER_W * 16,), jnp.int32),
            pltpu.VMEM((TC_PER_W * 16,), jnp.float32),
            pltpu.VMEM((H * 256,), jnp.int32),
            pltpu.VMEM((H * 256,), jnp.float32),
            pltpu.VMEM((H * N_OUT * 32,), jnp.float32),
            pltpu.VMEM((TC_PER_W, 16), jnp.float32),
            pltpu.SemaphoreType.DMA,
        ],
    )
    return kern(addr_rows, key2, ram_flat, wout_flat, wcnt_flat, rout_flat)


@jax.jit
def kernel(x_bits, conn_state, hash_coeff, conn_out, ram_state, ram_out):
    oh = (conn_state[..., None] == jnp.arange(C)[None, None, None, :])
    w_all = jnp.sum(hash_coeff[..., None] * oh.astype(jnp.int32), axis=2)
    w2 = jnp.transpose(w_all[:, :, :INPUT_BITS] % HASH,
                       (2, 0, 1)).reshape(INPUT_BITS, H * N_STATE)
    ws_t = jnp.transpose(w_all[:, :, INPUT_BITS:] % HASH, (0, 2, 1))
    pow2 = jnp.asarray(2 ** np.arange(N_BITS_OUT - 1, -1, -1), dtype=jnp.int32)
    oh_out = (conn_out[..., None] == jnp.arange(N_STATE)[None, None, None, :])
    wout = jnp.sum(pow2[None, None, :, None] * oh_out.astype(jnp.int32), axis=2)
    wout_t = jnp.transpose(wout, (0, 2, 1)).reshape(-1)
    wcnt_t = jnp.transpose(jnp.sum(oh_out, axis=2).astype(jnp.float32),
                           (0, 2, 1)).reshape(-1)

    sgn = (jax.nn.sigmoid(ram_state) > 0.5).astype(jnp.int32)
    sgn = sgn.reshape(H, N_STATE, 32, 2048)
    sp = jnp.sum(sgn << jnp.arange(32, dtype=jnp.int32)[None, None, :, None],
                 axis=2)
    sp_flat = sp.reshape(H, N_STATE * 2048)

    ain2, key2 = _phase_a(x_bits, w2)

    addr_hist = _phase_b(ain2, sp_flat, ws_t.reshape(H, N_STATE * 16))

    return _phase_c(addr_hist, key2, ram_state.reshape(-1), wout_t, wcnt_t,
                    ram_out.reshape(-1))


# speedup vs baseline: 66.7869x; 1.0948x over previous
import jax
import jax.numpy as jnp
import numpy as np
from jax import lax
from jax.experimental import pallas as pl
from jax.experimental.pallas import tpu as pltpu
from jax.experimental.pallas import tpu_sc as plsc

H = 16
INPUT_BITS = 24
N_STATE = 16
N_OUT = 16
N_BITS_OUT = 5
HASH = 65536
K_BITS = 4
T = 8192
C = INPUT_BITS + N_STATE

TB = 512
CH = 512
TC_PER_W = T // 32


def _phase_a_body(x_ref, w_ref, ain_ref, key_ref):
    x = x_ref[...]
    acc = jnp.zeros((TB, H * N_STATE), jnp.int32)
    for b in range(INPUT_BITS):
        acc = acc + x[:, b:b + 1] * w_ref[b:b + 1, :]
    ain_ref[...] = acc & (HASH - 1)
    k = x[:, 0:1] * 8 + x[:, 1:2] * 4 + x[:, 2:3] * 2 + x[:, 3:4]
    key_ref[...] = jnp.broadcast_to(k, (TB, 128))


def _phase_a(x_bits, w2):
    return pl.pallas_call(
        _phase_a_body,
        grid=(T // TB,),
        in_specs=[
            pl.BlockSpec((TB, INPUT_BITS), lambda t: (t, 0)),
            pl.BlockSpec((INPUT_BITS, H * N_STATE), lambda t: (0, 0)),
        ],
        out_specs=[
            pl.BlockSpec((TB, H * N_STATE), lambda t: (t, 0)),
            pl.BlockSpec((TB, 128), lambda t: (t, 0)),
        ],
        out_shape=[
            jax.ShapeDtypeStruct((T, H * N_STATE), jnp.int32),
            jax.ShapeDtypeStruct((T, 128), jnp.int32),
        ],
    )(x_bits, w2)


def _iota16():
    return lax.iota(jnp.int32, 16)


def _bcast_lane(vec, lane):
    idx = jnp.full((16,), lane, jnp.int32)
    return vec.at[idx].get(mode="promise_in_bounds")


def _phase_b_body(ain_hbm, sp_hbm, ws_hbm, addr_hbm, sp_v, ws_v,
                  ain_a, ain_b, addr_a, addr_b, sem_in, sem_out):
    c = lax.axis_index("c")
    s = lax.axis_index("s")
    h = c * 8 + s

    @pl.when(s < 8)
    def _():
        pltpu.sync_copy(sp_hbm.at[h], sp_v)
        pltpu.sync_copy(ws_hbm.at[h], ws_v)
        iota = _iota16()
        nbase = iota * 2048
        ain_bufs = (ain_a, ain_b)
        addr_bufs = (addr_a, addr_b)
        n_chunks = T // CH

        def make_step(ain_v, addr_v):
            def step(i, bits):
                acc = jnp.zeros((16,), jnp.int32)
                for si in range(N_STATE):
                    acc = acc + ws_v[pl.ds(si * 16, 16)] * _bcast_lane(bits, si)
                row = jnp.full((16,), i, jnp.int32)
                a = (plsc.load_gather(ain_v, [row, iota]) + acc) & (HASH - 1)
                w = plsc.load_gather(sp_v, [(a & 2047) + nbase])
                bit = (w >> (a >> 11)) & 1
                plsc.store_scatter(addr_v, [row, iota], a | (bit << 16))
                return bit
            return step

        def fetch(ci, buf):
            return pltpu.async_copy(
                ain_hbm.at[pl.ds(ci * CH, CH), pl.ds(h * 16, 16)], buf, sem_in)

        bits = jnp.zeros((16,), jnp.int32)
        in_d = fetch(0, ain_bufs[0])
        out_d = [None, None]
        for ci in range(n_chunks):
            b = ci % 2
            in_d.wait()
            if ci + 1 < n_chunks:
                in_d = fetch(ci + 1, ain_bufs[1 - b])
            if out_d[b] is not None:
                out_d[b].wait()
            bits = lax.fori_loop(0, CH, make_step(ain_bufs[b], addr_bufs[b]),
                                 bits, unroll=2)
            out_d[b] = pltpu.async_copy(
                addr_bufs[b], addr_hbm.at[pl.ds(h * T + ci * CH, CH), :],
                sem_out)
        out_d[0].wait()
        out_d[1].wait()


def _phase_b(ain2, sp_flat, ws_flat):
    kern = pl.kernel(
        _phase_b_body,
        out_type=jax.ShapeDtypeStruct((H * T, 16), jnp.int32),
        mesh=plsc.VectorSubcoreMesh(core_axis_name="c", subcore_axis_name="s"),
        compiler_params=pltpu.CompilerParams(needs_layout_passes=False,
                                             use_tc_tiling_on_sc=False),
        scratch_types=[
            pltpu.VMEM((N_STATE * 2048,), jnp.int32),
            pltpu.VMEM((N_STATE * 16,), jnp.int32),
            pltpu.VMEM((CH, 16), jnp.int32),
            pltpu.VMEM((CH, 16), jnp.int32),
            pltpu.VMEM((CH, 16), jnp.int32),
            pltpu.VMEM((CH, 16), jnp.int32),
            pltpu.SemaphoreType.DMA,
            pltpu.SemaphoreType.DMA,
        ],
    )
    return kern(ain2, sp_flat, ws_flat)


def _phase_c_body(addr_hbm, key_hbm, ram_hbm, wout_hbm, wcnt_hbm,
                  rout_hbm, out_hbm, key_v, ridx_v, addr_v, gidx_v, raw_v,
                  wout_v, wcnt_v, rout_v, out_v, sem):
    c = lax.axis_index("c")
    s = lax.axis_index("s")
    wid = s * 2 + c
    t0 = wid * TC_PER_W
    iota = _iota16()
    zeros = jnp.zeros((16,), jnp.int32)

    pltpu.sync_copy(wout_hbm, wout_v)
    pltpu.sync_copy(wcnt_hbm, wcnt_v)
    pltpu.sync_copy(rout_hbm, rout_v)
    pltpu.sync_copy(key_hbm.at[pl.ds(t0, TC_PER_W), pl.ds(0, 16)], key_v)

    def mk_ridx(g, _):
        kvec = plsc.load_gather(key_v, [g * 16 + iota, zeros])
        ridx_v[pl.ds(g * 16, 16)] = kvec * T + (t0 + g * 16 + iota)
        return 0

    lax.fori_loop(0, TC_PER_W // 16, mk_ridx, 0)

    addr_descs = [
        pltpu.async_copy(addr_hbm.at[ridx_v.at[pl.ds(g * 128, 128)]],
                         addr_v.at[pl.ds(g * 128, 128)], sem)
        for g in range(TC_PER_W // 128)]
    for d in addr_descs:
        d.wait()

    def mk_gidx(i, _):
        row = jnp.full((16,), i, jnp.int32)
        k_s = plsc.load_gather(key_v, [row, iota])
        a = plsc.load_gather(addr_v, [row, iota])
        gidx_v[pl.ds(i * 16, 16)] = (k_s * 16 + iota) * HASH + (a & (HASH - 1))
        return 0

    lax.fori_loop(0, TC_PER_W, mk_gidx, 0)

    n_g = TC_PER_W * 16 // 128
    raw_descs = [
        pltpu.async_copy(ram_hbm.at[gidx_v.at[pl.ds(j * 128, 128)]],
                         raw_v.at[pl.ds(j * 128, 128)], sem)
        for j in range(n_g)]
    for d in raw_descs:
        d.wait()

    def emit(i, _):
        row = jnp.full((16,), i, jnp.int32)
        k_s = plsc.load_gather(key_v, [row, iota])
        v = plsc.load_gather(addr_v, [row, iota])
        hard = (v >> 16) & 1
        raw = raw_v[pl.ds(i * 16, 16)]
        sig = 1.0 / (1.0 + jnp.exp(-raw))
        acc_a = jnp.zeros((16,), jnp.int32)
        acc_c = jnp.zeros((16,), jnp.float32)
        base = k_s * 256 + iota
        for si in range(N_STATE):
            wo = plsc.load_gather(wout_v, [base + si * 16])
            wc = plsc.load_gather(wcnt_v, [base + si * 16])
            acc_a = acc_a + wo * _bcast_lane(hard, si)
            acc_c = acc_c + wc * _bcast_lane(sig, si)
        oidx = k_s * 512 + iota * 32 + acc_a
        lut = plsc.load_gather(rout_v, [oidx])
        plsc.store_scatter(out_v, [row, iota],
                           lut * (0.5 + acc_c * (1.0 / N_BITS_OUT)))
        return 0

    lax.fori_loop(0, TC_PER_W, emit, 0)
    pltpu.sync_copy(out_v, out_hbm.at[pl.ds(t0, TC_PER_W), :])


def _phase_c(addr_rows, key2, ram_flat, wout_flat, wcnt_flat, rout_flat):
    kern = pl.kernel(
        _phase_c_body,
        out_type=jax.ShapeDtypeStruct((T, 16), jnp.float32),
        mesh=plsc.VectorSubcoreMesh(core_axis_name="c", subcore_axis_name="s"),
        compiler_params=pltpu.CompilerParams(needs_layout_passes=False,
                                             use_tc_tiling_on_sc=False),
        scratch_types=[
            pltpu.VMEM((TC_PER_W, 16), jnp.int32),
            pltpu.VMEM((TC_PER_W,), jnp.int32),
            pltpu.VMEM((TC_PER_W, 16), jnp.int32),
            pltpu.VMEM((TC_PER_W * 16,), jnp.int32),
            pltpu.VMEM((TC_PER_W * 16,), jnp.float32),
            pltpu.VMEM((H * 256,), jnp.int32),
            pltpu.VMEM((H * 256,), jnp.float32),
            pltpu.VMEM((H * N_OUT * 32,), jnp.float32),
            pltpu.VMEM((TC_PER_W, 16), jnp.float32),
            pltpu.SemaphoreType.DMA,
        ],
    )
    return kern(addr_rows, key2, ram_flat, wout_flat, wcnt_flat, rout_flat)


@jax.jit
def kernel(x_bits, conn_state, hash_coeff, conn_out, ram_state, ram_out):
    oh = (conn_state[..., None] == jnp.arange(C)[None, None, None, :])
    w_all = jnp.sum(hash_coeff[..., None] * oh.astype(jnp.int32), axis=2)
    w2 = jnp.transpose(w_all[:, :, :INPUT_BITS] % HASH,
                       (2, 0, 1)).reshape(INPUT_BITS, H * N_STATE)
    ws_t = jnp.transpose(w_all[:, :, INPUT_BITS:] % HASH, (0, 2, 1))
    pow2 = jnp.asarray(2 ** np.arange(N_BITS_OUT - 1, -1, -1), dtype=jnp.int32)
    oh_out = (conn_out[..., None] == jnp.arange(N_STATE)[None, None, None, :])
    wout = jnp.sum(pow2[None, None, :, None] * oh_out.astype(jnp.int32), axis=2)
    wout_t = jnp.transpose(wout, (0, 2, 1)).reshape(-1)
    wcnt_t = jnp.transpose(jnp.sum(oh_out, axis=2).astype(jnp.float32),
                           (0, 2, 1)).reshape(-1)

    sgn = (jax.nn.sigmoid(ram_state) > 0.5).astype(jnp.int32)
    sgn = sgn.reshape(H, N_STATE, 32, 2048)
    sp = jnp.sum(sgn << jnp.arange(32, dtype=jnp.int32)[None, None, :, None],
                 axis=2)
    sp_flat = sp.reshape(H, N_STATE * 2048)

    ain2, key2 = _phase_a(x_bits, w2)

    addr_hist = _phase_b(ain2, sp_flat, ws_t.reshape(H, N_STATE * 16))

    return _phase_c(addr_hist, key2, ram_state.reshape(-1), wout_t, wcnt_t,
                    ram_out.reshape(-1))
